# baseline (speedup 1.0000x reference)
import functools

import jax
import jax.numpy as jnp
from jax import lax
from jax.experimental import pallas as pl
from jax.experimental.pallas import tpu as pltpu
from jax.experimental.pallas import tpu_sc as plsc

NUM_EMBEDDINGS = 1000000
EMBEDDING_DIM = 64
BATCH = 16384
FIELDS = 26

B_TOTAL = BATCH * FIELDS
NW = 32
B_PER_W = B_TOTAL // NW
CHUNK = 1024
N_CHUNKS = B_PER_W // CHUNK


def _make_gather_kernel():
    mesh = plsc.VectorSubcoreMesh(core_axis_name="c", subcore_axis_name="s")

    @functools.partial(
        pl.kernel,
        mesh=mesh,
        out_type=jax.ShapeDtypeStruct((B_TOTAL, EMBEDDING_DIM), jnp.float32),
        compiler_params=pltpu.CompilerParams(use_tc_tiling_on_sc=False),
        scratch_types=[
            pltpu.VMEM((CHUNK,), jnp.int32),
            pltpu.VMEM((CHUNK, EMBEDDING_DIM), jnp.float32),
            pltpu.SemaphoreType.DMA,
        ],
    )
    def gather_kernel(table_hbm, idx_hbm, out_hbm, idx_v, rows_v, sem):
        wid = lax.axis_index("s") * 2 + lax.axis_index("c")
        w_base = wid * B_PER_W

        def body(i, carry):
            base = w_base + i * CHUNK
            pltpu.sync_copy(idx_hbm.at[pl.ds(base, CHUNK)], idx_v)
            pltpu.async_copy(table_hbm.at[idx_v], rows_v, sem).wait()
            pltpu.sync_copy(rows_v, out_hbm.at[pl.ds(base, CHUNK)])
            return carry

        lax.fori_loop(0, N_CHUNKS, body, 0)

    return gather_kernel


_gather = _make_gather_kernel()


@jax.jit
def kernel(input_indices, weight):
    idx_flat = input_indices.reshape(B_TOTAL)
    out_flat = _gather(weight, idx_flat)
    return out_flat.reshape(BATCH, FIELDS, EMBEDDING_DIM)


# speedup vs baseline: 1.0063x; 1.0063x over previous
import functools

import jax
import jax.numpy as jnp
from jax import lax
from jax.experimental import pallas as pl
from jax.experimental.pallas import tpu as pltpu
from jax.experimental.pallas import tpu_sc as plsc

NUM_EMBEDDINGS = 1000000
EMBEDDING_DIM = 64
BATCH = 16384
FIELDS = 26

B_TOTAL = BATCH * FIELDS
NW = 32
B_PER_W = B_TOTAL // NW
CHUNK = 832
N_CHUNKS = B_PER_W // CHUNK


def _make_gather_kernel():
    mesh = plsc.VectorSubcoreMesh(core_axis_name="c", subcore_axis_name="s")

    @functools.partial(
        pl.kernel,
        mesh=mesh,
        out_type=jax.ShapeDtypeStruct((B_TOTAL, EMBEDDING_DIM), jnp.float32),
        compiler_params=pltpu.CompilerParams(use_tc_tiling_on_sc=False),
        scratch_types=[
            pltpu.VMEM((CHUNK,), jnp.int32),
            pltpu.VMEM((CHUNK,), jnp.int32),
            pltpu.VMEM((CHUNK, EMBEDDING_DIM), jnp.float32),
            pltpu.VMEM((CHUNK, EMBEDDING_DIM), jnp.float32),
            pltpu.SemaphoreType.DMA,
            pltpu.SemaphoreType.DMA,
            pltpu.SemaphoreType.DMA,
            pltpu.SemaphoreType.DMA,
        ],
    )
    def gather_kernel(table_hbm, idx_hbm, out_hbm,
                      idx0, idx1, rows0, rows1,
                      gsem0, gsem1, wsem0, wsem1):
        wid = lax.axis_index("s") * 2 + lax.axis_index("c")
        w_base = wid * B_PER_W

        idx_v = (idx0, idx1)
        rows_v = (rows0, rows1)
        gsem = (gsem0, gsem1)
        wsem = (wsem0, wsem1)
        gathers = [None, None]
        writes = [None, None]

        for i in range(N_CHUNKS):
            b = i % 2
            base = w_base + i * CHUNK
            if writes[b] is not None:
                writes[b].wait()
            pltpu.sync_copy(idx_hbm.at[pl.ds(base, CHUNK)], idx_v[b])
            gathers[b] = pltpu.async_copy(table_hbm.at[idx_v[b]], rows_v[b], gsem[b])
            if i >= 1:
                pb = (i - 1) % 2
                pbase = w_base + (i - 1) * CHUNK
                gathers[pb].wait()
                writes[pb] = pltpu.async_copy(
                    rows_v[pb], out_hbm.at[pl.ds(pbase, CHUNK)], wsem[pb])

        last = N_CHUNKS - 1
        lb = last % 2
        gathers[lb].wait()
        writes[lb] = pltpu.async_copy(
            rows_v[lb], out_hbm.at[pl.ds(w_base + last * CHUNK, CHUNK)], wsem[lb])
        writes[0].wait()
        writes[1].wait()

    return gather_kernel


_gather = _make_gather_kernel()


@jax.jit
def kernel(input_indices, weight):
    idx_flat = input_indices.reshape(B_TOTAL)
    out_flat = _gather(weight, idx_flat)
    return out_flat.reshape(BATCH, FIELDS, EMBEDDING_DIM)
